# baseline (speedup 1.0000x reference)
import functools
import numpy as np
import jax
import jax.numpy as jnp
from jax.experimental import pallas as pl
from jax.experimental.pallas import tpu as pltpu

NB = 4


def _round_up(x, m):
    return ((x + m - 1) // m) * m


def _cdiv(a, b):
    return -(-a // b)


def _tap_kernel(w_ref, x_ref, sc_ref, sf_ref, im_ref, om_ref,
                y_ref, s_ref, q_ref, *scratch,
                offsets, mp, gi, apply_in):
    om = om_ref[...]
    cp = w_ref.shape[1]

    def body(i, carry):
        sa, qa = carry
        if apply_in:
            xt_ref = scratch[0]
            xr = x_ref[i].astype(jnp.float32)
            xt_ref[...] = (jnp.maximum(xr * sc_ref[...] + sf_ref[...], 0.0)
                           * im_ref[...]).astype(jnp.bfloat16)
        acc = None
        for t, off in enumerate(offsets):
            rhs = (scratch[0][:, off:off + mp] if apply_in
                   else x_ref[i, :, off:off + mp])
            c = jnp.dot(w_ref[t], rhs, preferred_element_type=jnp.float32)
            acc = c if acc is None else acc + c
        y_ref[i] = acc.astype(y_ref.dtype)
        masked = acc * om
        return (sa + jnp.sum(masked, axis=1, keepdims=True),
                qa + jnp.sum(masked * acc, axis=1, keepdims=True))

    z = jnp.zeros((cp, 1), jnp.float32)
    s, q = jax.lax.fori_loop(0, gi, body, (z, z))
    s_ref[0] = s
    q_ref[0] = q


def _tap_layer(wt, xs, sc, sf, im, om, mp, offsets, apply_in):
    N, Cs, Li = xs.shape
    T, Cp, _ = wt.shape
    gi = N // NB
    kern = functools.partial(_tap_kernel, offsets=tuple(offsets), mp=mp, gi=gi,
                             apply_in=apply_in)
    mrows = im.shape[0]
    scratch = [pltpu.VMEM((Cs, Li), jnp.bfloat16)] if apply_in else []
    blk = (T * Cp * Cs * 2 + gi * Cs * Li * 2 + gi * Cp * mp * 2
           + Cp * mp * 4 + mrows * Li * 4 + mp * 4 + Cs * Li * 2)
    vmem = int(min(max(3 * blk, 32 * 1024 * 1024), 56 * 1024 * 1024))
    y, s, q = pl.pallas_call(
        kern,
        out_shape=(jax.ShapeDtypeStruct((N, Cp, mp), jnp.bfloat16),
                   jax.ShapeDtypeStruct((NB, Cp, 1), jnp.float32),
                   jax.ShapeDtypeStruct((NB, Cp, 1), jnp.float32)),
        grid=(NB,),
        in_specs=[pl.BlockSpec((T, Cp, Cs), lambda n: (0, 0, 0)),
                  pl.BlockSpec((gi, Cs, Li), lambda n: (n, 0, 0)),
                  pl.BlockSpec((Cs, 1), lambda n: (0, 0)),
                  pl.BlockSpec((Cs, 1), lambda n: (0, 0)),
                  pl.BlockSpec((mrows, Li), lambda n: (0, 0)),
                  pl.BlockSpec((1, mp), lambda n: (0, 0))],
        out_specs=(pl.BlockSpec((gi, Cp, mp), lambda n: (n, 0, 0)),
                   pl.BlockSpec((1, Cp, 1), lambda n: (n, 0, 0)),
                   pl.BlockSpec((1, Cp, 1), lambda n: (n, 0, 0))),
        scratch_shapes=scratch,
        compiler_params=pltpu.CompilerParams(
            dimension_semantics=("parallel",), vmem_limit_bytes=vmem),
    )(wt, xs, sc, sf, im, om)
    return y, s[:, :, 0], q[:, :, 0]


def _bnrelu_kernel(y_ref, sc_ref, sf_ref, o_ref):
    o_ref[...] = jnp.maximum(
        y_ref[...].astype(jnp.float32) * sc_ref[...] + sf_ref[...], 0.0)


def _bnrelu(y, sc, sf):
    N, Cp, Mp = y.shape
    gi = N // NB
    return pl.pallas_call(
        _bnrelu_kernel,
        out_shape=jax.ShapeDtypeStruct((N, Cp, Mp), jnp.float32),
        grid=(NB,),
        in_specs=[pl.BlockSpec((gi, Cp, Mp), lambda n: (n, 0, 0)),
                  pl.BlockSpec((Cp, 1), lambda n: (0, 0)),
                  pl.BlockSpec((Cp, 1), lambda n: (0, 0))],
        out_specs=pl.BlockSpec((gi, Cp, Mp), lambda n: (n, 0, 0)),
        compiler_params=pltpu.CompilerParams(
            dimension_semantics=("parallel",)),
    )(y, sc.reshape(Cp, 1), sf.reshape(Cp, 1))


def _fc_kernel(x_ref, sc_ref, sf_ref, w1, b1, w2, b2, w3, b3, w4, b4, w5, b5,
               w6, b6, h_ref, d_ref):
    xr = x_ref[...].astype(jnp.float32)
    z = jnp.maximum(xr * sc_ref[...] + sf_ref[...], 0.0)

    def lin(v, w, b):
        return jnp.dot(v.astype(jnp.bfloat16), w[...],
                       preferred_element_type=jnp.float32) + b[...]

    z = jnp.maximum(lin(z, w1, b1), 0.0)
    z = jnp.maximum(lin(z, w2, b2), 0.0)
    h = jax.nn.sigmoid(lin(z, w3, b3))
    h_ref[...] = h
    d = jnp.maximum(lin(h, w4, b4), 0.0)
    d = jnp.maximum(lin(d, w5, b5), 0.0)
    d = jnp.maximum(lin(d, w6, b6), 0.0)
    d_ref[...] = d


def _fc_stack(flat_raw, sc_flat, sf_flat, p):
    Nb, F = flat_raw.shape
    names = ["en_lin1", "en_lin2", "en_lin3", "de_lin3", "de_lin2", "de_lin1"]
    args = [flat_raw, sc_flat, sf_flat]
    for n in names:
        args.append(p[n + "_w"].T.astype(jnp.bfloat16))
        args.append(p[n + "_b"].reshape(1, -1).astype(jnp.float32))
    h_dim = p["en_lin3_w"].shape[0]
    h, d = pl.pallas_call(
        _fc_kernel,
        out_shape=(jax.ShapeDtypeStruct((Nb, h_dim), jnp.float32),
                   jax.ShapeDtypeStruct((Nb, F), jnp.float32)),
    )(*args)
    return h, d


def _fold_conv_weight(w, sh, sw):
    C_out, C_in, kd, kh, kw = w.shape
    Uh, Uw = _cdiv(kh, sh), _cdiv(kw, sw)
    wp = jnp.pad(w, ((0, 0), (0, 0), (0, 0), (0, Uh * sh - kh), (0, Uw * sw - kw)))
    wp = wp.reshape(C_out, C_in, kd, Uh, sh, Uw, sw)
    wp = wp.transpose(2, 3, 5, 0, 1, 4, 6)
    return wp.reshape(kd * Uh * Uw, C_out, C_in * sh * sw)


def _tconv_axis_plan(k, s, p, n_out):
    q_cnt = _cdiv(n_out, s)
    j0 = [(k - 1 - p - r) % s for r in range(s)]
    base = [(r + j0[r] - (k - 1) + p) // s for r in range(s)]
    L = [(k - 1 - j0[r]) // s + 1 for r in range(s)]
    bmin = min(base)
    U = max(base[r] - bmin + L[r] for r in range(s))
    padL = -bmin
    J = np.full((s, U), -1, dtype=np.int64)
    for r in range(s):
        off = base[r] - bmin
        for u in range(L[r]):
            J[r, off + u] = k - 1 - j0[r] - s * u
    return padL, U, J, q_cnt


def _expand_tconv_weight(w, Jd, Jh, Jw):
    sd, Ud = Jd.shape
    sh, Uh = Jh.shape
    sw, Uw = Jw.shape
    jd = Jd.T.reshape(Ud, 1, 1, sd, 1, 1)
    jh = Jh.T.reshape(1, Uh, 1, 1, sh, 1)
    jw = Jw.T.reshape(1, 1, Uw, 1, 1, sw)
    jd, jh, jw = np.broadcast_arrays(jd, jh, jw)
    valid = (jd >= 0) & (jh >= 0) & (jw >= 0)
    wp = w.transpose(1, 0, 2, 3, 4)
    g = wp[:, :, np.maximum(jd, 0), np.maximum(jh, 0), np.maximum(jw, 0)]
    g = g * jnp.asarray(valid, wp.dtype)
    g = g.transpose(2, 3, 4, 0, 5, 6, 7, 1)
    return g.reshape(Ud * Uh * Uw, g.shape[3] * sd * sh * sw, g.shape[-1])


def _bn_affine(ssum, qsum, count, gamma, beta, eps=1e-5):
    mean = ssum / count
    var = jnp.maximum(qsum / count - mean * mean, 0.0)
    scale = gamma.astype(jnp.float32) / jnp.sqrt(var + eps)
    shift = beta.astype(jnp.float32) - mean * scale
    return scale, shift


def _valid_mask(grid_dims, valid_dims, mp):
    dg, hg, wg = grid_dims
    do, ho, wo = valid_dims
    m = np.zeros((dg, hg, wg), np.float32)
    m[:do, :ho, :wo] = 1.0
    out = np.zeros((1, mp), np.float32)
    out[0, :m.size] = m.reshape(-1)
    return jnp.asarray(out)


def _conv_in_mask(C_in, sh, sw, grid_dims, pads, in_dims, li):
    Dg, Hq, Wq = grid_dims
    pd, ph, pw = pads
    D, H, W = in_dims
    d = np.arange(Dg).reshape(-1, 1, 1)
    h = np.arange(Hq).reshape(1, -1, 1)
    w = np.arange(Wq).reshape(1, 1, -1)
    rows = []
    for a in range(sh):
        for b in range(sw):
            ok = ((d >= pd) & (d < pd + D)
                  & (h * sh + a >= ph) & (h * sh + a < ph + H)
                  & (w * sw + b >= pw) & (w * sw + b < pw + W))
            rows.append(ok.reshape(-1))
    m = np.stack(rows).astype(np.float32)
    out = np.zeros((C_in * sh * sw, li), np.float32)
    out[:, :m.shape[1]] = np.tile(m, (C_in, 1))
    return jnp.asarray(out)


def _deconv_in_mask(grid_dims, padl, in_dims, li):
    Dg, Hg, Wg = grid_dims
    pld, plh, plw = padl
    Di, Hi, Wi = in_dims
    m = np.zeros((Dg, Hg, Wg), np.float32)
    m[pld:pld + Di, plh:plh + Hi, plw:plw + Wi] = 1.0
    out = np.zeros((1, li), np.float32)
    out[0, :m.size] = m.reshape(-1)
    return jnp.asarray(out)


ENC_CFG = [("en_conv1", "en_norm1", (1, 3, 3), (0, 2, 2)),
           ("en_conv2", "en_norm2", (1, 2, 2), (0, 2, 2)),
           ("en_conv3", "en_norm3", (1, 2, 2), (0, 2, 2)),
           ("en_conv4", "en_norm4", (1, 2, 2), (0, 2, 2))]
DEC_CFG = [("de_conv4", "de_norm4", (1, 2, 2), (0, 2, 2)),
           ("de_conv3", "de_norm3", (1, 2, 2), (0, 2, 2)),
           ("de_conv2", "de_norm2", (1, 2, 2), (0, 2, 2)),
           ("de_conv1", "de_norm1", (1, 3, 3), (0, 2, 2))]


def _enc_layer(x5_raw, aff, w, gamma, beta, stride, pad):
    N, C_in, D, H, W = x5_raw.shape
    C_out, _, kd, kh, kw = w.shape
    _, sh, sw = stride
    pd, ph, pw = pad
    Do = D + 2 * pd - kd + 1
    Ho = (H + 2 * ph - kh) // sh + 1
    Wo = (W + 2 * pw - kw) // sw + 1
    Uh, Uw = _cdiv(kh, sh), _cdiv(kw, sw)
    Dg = D + 2 * pd
    Hq = max(_cdiv(H + 2 * ph, sh), Ho + Uh - 1)
    Wq = max(_cdiv(W + 2 * pw, sw), Wo + Uw - 1)

    Mv = Dg * Hq * Wq

    def _fold(chunk):
        cpad = jnp.pad(chunk, ((0, 0), (0, 0), (pd, Dg - D - pd),
                               (ph, Hq * sh - H - ph), (pw, Wq * sw - W - pw)))
        cs = cpad.reshape(-1, C_in, Dg, Hq, sh, Wq, sw).transpose(0, 1, 4, 6, 2, 3, 5)
        return cs.reshape(-1, C_in * sh * sw, Mv)

    if aff is None and N >= 16:
        xs = jnp.concatenate([_fold(x5_raw[i:i + N // 8])
                              for i in range(0, N, N // 8)], axis=0)
    else:
        xs = _fold(x5_raw)

    offsets = tuple(td * Hq * Wq + th * Wq + tw
                    for td in range(kd) for th in range(Uh) for tw in range(Uw))
    Mp = _round_up(Mv, 128)
    Li = Mp + (_round_up(offsets[-1], 128) if offsets[-1] else 0)
    xs = jnp.pad(xs, ((0, 0), (0, 0), (0, Li - Mv)))

    wt = _fold_conv_weight(w.astype(jnp.bfloat16), sh, sw)
    Cp = _round_up(C_out, 8)
    wt = jnp.pad(wt, ((0, 0), (0, Cp - C_out), (0, 0)))
    Cs = C_in * sh * sw

    if aff is not None:
        sc = jnp.repeat(aff[0], sh * sw).reshape(Cs, 1)
        sf = jnp.repeat(aff[1], sh * sw).reshape(Cs, 1)
        im = _conv_in_mask(C_in, sh, sw, (Dg, Hq, Wq), pad, (D, H, W), Li)
    else:
        sc = jnp.ones((Cs, 1), jnp.float32)
        sf = jnp.zeros((Cs, 1), jnp.float32)
        im = jnp.ones((1, Li), jnp.float32)
    om = _valid_mask((Dg, Hq, Wq), (Do, Ho, Wo), Mp)

    y, s, q = _tap_layer(wt, xs, sc, sf, im, om, Mp, offsets, aff is not None)
    cnt = float(N * Do * Ho * Wo)
    scale, shift = _bn_affine(jnp.sum(s[:, :C_out], 0), jnp.sum(q[:, :C_out], 0),
                              cnt, gamma, beta)
    y5_raw = (y[:, :C_out, :Mv].reshape(N, C_out, Dg, Hq, Wq)
              [:, :, :Do, :Ho, :Wo])
    return y5_raw, (scale, shift)


def _dec_layer(x5_raw, aff, w, gamma, beta, stride, pad):
    N, C_in, Di, Hi, Wi = x5_raw.shape
    _, C_out, kd, kh, kw = w.shape
    sd, sh, sw = stride
    pd, ph, pw = pad
    Do = (Di - 1) * sd - 2 * pd + kd
    Ho = (Hi - 1) * sh - 2 * ph + kh
    Wo = (Wi - 1) * sw - 2 * pw + kw
    pld, Ud, Jd, Qd = _tconv_axis_plan(kd, sd, pd, Do)
    plh, Uh, Jh, Qh = _tconv_axis_plan(kh, sh, ph, Ho)
    plw, Uw, Jw, Qw = _tconv_axis_plan(kw, sw, pw, Wo)
    Dg, Hg, Wg = Qd + Ud - 1, Qh + Uh - 1, Qw + Uw - 1

    xpad = jnp.pad(x5_raw, ((0, 0), (0, 0), (pld, Dg - Di - pld),
                            (plh, Hg - Hi - plh), (plw, Wg - Wi - plw)))
    Mv = Dg * Hg * Wg
    xs = xpad.reshape(N, C_in, Mv)

    offsets = tuple(td * Hg * Wg + th * Wg + tw
                    for td in range(Ud) for th in range(Uh) for tw in range(Uw))
    Mp = _round_up(Mv, 128)
    Li = Mp + (_round_up(offsets[-1], 128) if offsets[-1] else 0)
    xs = jnp.pad(xs, ((0, 0), (0, 0), (0, Li - Mv)))

    nph = sd * sh * sw
    wt = _expand_tconv_weight(w.astype(jnp.bfloat16), Jd, Jh, Jw)
    Cph = C_out * nph
    Cp = _round_up(Cph, 8)
    wt = jnp.pad(wt, ((0, 0), (0, Cp - Cph), (0, 0)))

    if aff is not None:
        sc = aff[0].reshape(C_in, 1)
        sf = aff[1].reshape(C_in, 1)
        im = _deconv_in_mask((Dg, Hg, Wg), (pld, plh, plw), (Di, Hi, Wi), Li)
    else:
        sc = jnp.ones((C_in, 1), jnp.float32)
        sf = jnp.zeros((C_in, 1), jnp.float32)
        im = jnp.ones((1, Li), jnp.float32)
    om = _valid_mask((Dg, Hg, Wg), (Qd, Qh, Qw), Mp)

    y, s, q = _tap_layer(wt, xs, sc, sf, im, om, Mp, offsets, aff is not None)
    cnt = float(N * Do * Ho * Wo)
    ssum = jnp.sum(s[:, :Cph], 0).reshape(C_out, nph).sum(1)
    qsum = jnp.sum(q[:, :Cph], 0).reshape(C_out, nph).sum(1)
    scale, shift = _bn_affine(ssum, qsum, cnt, gamma, beta)
    geom = (Cph, Mv, C_out, (sd, sh, sw), (Dg, Hg, Wg), (Qd, Qh, Qw),
            (Do, Ho, Wo))
    return y, (scale, shift), geom


def _dec_to5d(y, geom, chunks=1):
    N = y.shape[0]
    Cph, Mv, C_out, (sd, sh, sw), (Dg, Hg, Wg), (Qd, Qh, Qw), (Do, Ho, Wo) = geom

    def _d2s(yc):
        y8 = yc[:, :Cph, :Mv].reshape(-1, C_out, sd, sh, sw, Dg, Hg, Wg)
        y8 = y8[:, :, :, :, :, :Qd, :Qh, :Qw]
        y8 = y8.transpose(0, 1, 5, 2, 6, 3, 7, 4)
        return y8.reshape(-1, C_out, Qd * sd, Qh * sh, Qw * sw)[:, :, :Do, :Ho, :Wo]

    if chunks > 1 and N % chunks == 0:
        g = N // chunks
        return jnp.concatenate([_d2s(y[i:i + g]) for i in range(0, N, g)], axis=0)
    return _d2s(y)


def kernel(x, en_conv1_w, en_norm1_g, en_norm1_b, en_conv2_w, en_norm2_g, en_norm2_b, en_conv3_w, en_norm3_g, en_norm3_b, en_conv4_w, en_norm4_g, en_norm4_b, de_conv4_w, de_norm4_g, de_norm4_b, de_conv3_w, de_norm3_g, de_norm3_b, de_conv2_w, de_norm2_g, de_norm2_b, de_conv1_w, de_norm1_g, de_norm1_b, en_lin1_w, en_lin1_b, en_lin2_w, en_lin2_b, en_lin3_w, en_lin3_b, de_lin3_w, de_lin3_b, de_lin2_w, de_lin2_b, de_lin1_w, de_lin1_b):
    p = {
        "en_conv1_w": en_conv1_w, "en_norm1_g": en_norm1_g, "en_norm1_b": en_norm1_b,
        "en_conv2_w": en_conv2_w, "en_norm2_g": en_norm2_g, "en_norm2_b": en_norm2_b,
        "en_conv3_w": en_conv3_w, "en_norm3_g": en_norm3_g, "en_norm3_b": en_norm3_b,
        "en_conv4_w": en_conv4_w, "en_norm4_g": en_norm4_g, "en_norm4_b": en_norm4_b,
        "de_conv4_w": de_conv4_w, "de_norm4_g": de_norm4_g, "de_norm4_b": de_norm4_b,
        "de_conv3_w": de_conv3_w, "de_norm3_g": de_norm3_g, "de_norm3_b": de_norm3_b,
        "de_conv2_w": de_conv2_w, "de_norm2_g": de_norm2_g, "de_norm2_b": de_norm2_b,
        "de_conv1_w": de_conv1_w, "de_norm1_g": de_norm1_g, "de_norm1_b": de_norm1_b,
        "en_lin1_w": en_lin1_w, "en_lin1_b": en_lin1_b,
        "en_lin2_w": en_lin2_w, "en_lin2_b": en_lin2_b,
        "en_lin3_w": en_lin3_w, "en_lin3_b": en_lin3_b,
        "de_lin3_w": de_lin3_w, "de_lin3_b": de_lin3_b,
        "de_lin2_w": de_lin2_w, "de_lin2_b": de_lin2_b,
        "de_lin1_w": de_lin1_w, "de_lin1_b": de_lin1_b,
    }
    N = x.shape[0]
    cur5 = x.astype(jnp.bfloat16)
    aff = None
    for conv, norm, stride, pad in ENC_CFG:
        cur5, aff = _enc_layer(cur5, aff, p[conv + "_w"], p[norm + "_g"],
                               p[norm + "_b"], stride, pad)

    enc_size = cur5.shape
    flat_raw = cur5.reshape(N, -1)
    rep = enc_size[2] * enc_size[3] * enc_size[4]
    sc_flat = jnp.repeat(aff[0], rep).reshape(1, -1)
    sf_flat = jnp.repeat(aff[1], rep).reshape(1, -1)
    h, d = _fc_stack(flat_raw, sc_flat, sf_flat, p)

    cur5 = d.reshape(enc_size).astype(jnp.bfloat16)
    aff = None
    for li, (conv, norm, stride, pad) in enumerate(DEC_CFG):
        y, aff_new, geom = _dec_layer(cur5, aff, p[conv + "_w"], p[norm + "_g"],
                                      p[norm + "_b"], stride, pad)
        if li < len(DEC_CFG) - 1:
            cur5 = _dec_to5d(y, geom)
            aff = aff_new
        else:
            Cph, nph = geom[0], geom[3][0] * geom[3][1] * geom[3][2]
            Cp = y.shape[1]
            sc_ph = jnp.pad(jnp.repeat(aff_new[0], nph), (0, Cp - Cph))
            sf_ph = jnp.pad(jnp.repeat(aff_new[1], nph), (0, Cp - Cph))
            o = _bnrelu(y, sc_ph, sf_ph)
            recon = _dec_to5d(o, geom, chunks=8)
    return h, recon


# speedup vs baseline: 1.2732x; 1.2732x over previous
import functools
import numpy as np
import jax
import jax.numpy as jnp
from jax.experimental import pallas as pl
from jax.experimental.pallas import tpu as pltpu

NB = 8


def _round_up(x, m):
    return ((x + m - 1) // m) * m


def _cdiv(a, b):
    return -(-a // b)


def _tap_kernel(w_ref, x_ref, sc_ref, sf_ref, im_ref, om_ref,
                y_ref, s_ref, q_ref, *scratch,
                offsets, mp, gi, apply_in):
    om = om_ref[...]
    cp = w_ref.shape[1]

    def body(i, carry):
        sa, qa = carry
        if apply_in:
            xt_ref = scratch[0]
            xr = x_ref[i].astype(jnp.float32)
            xt_ref[...] = (jnp.maximum(xr * sc_ref[...] + sf_ref[...], 0.0)
                           * im_ref[...]).astype(jnp.bfloat16)
        acc = None
        for t, off in enumerate(offsets):
            rhs = (scratch[0][:, off:off + mp] if apply_in
                   else x_ref[i, :, off:off + mp])
            c = jnp.dot(w_ref[t], rhs, preferred_element_type=jnp.float32)
            acc = c if acc is None else acc + c
        y_ref[i] = acc.astype(y_ref.dtype)
        masked = acc * om
        return (sa + jnp.sum(masked, axis=1, keepdims=True),
                qa + jnp.sum(masked * acc, axis=1, keepdims=True))

    z = jnp.zeros((cp, 1), jnp.float32)
    s, q = jax.lax.fori_loop(0, gi, body, (z, z))
    s_ref[0] = s
    q_ref[0] = q


def _tap_layer(wt, xs, sc, sf, im, om, mp, offsets, apply_in):
    N, Cs, Li = xs.shape
    T, Cp, _ = wt.shape
    gi = N // NB
    kern = functools.partial(_tap_kernel, offsets=tuple(offsets), mp=mp, gi=gi,
                             apply_in=apply_in)
    mrows = im.shape[0]
    scratch = [pltpu.VMEM((Cs, Li), jnp.bfloat16)] if apply_in else []
    blk = (T * Cp * Cs * 2 + gi * Cs * Li * 2 + gi * Cp * mp * 2
           + Cp * mp * 4 + mrows * Li * 4 + mp * 4 + Cs * Li * 2)
    vmem = int(min(max(3 * blk, 32 * 1024 * 1024), 56 * 1024 * 1024))
    y, s, q = pl.pallas_call(
        kern,
        out_shape=(jax.ShapeDtypeStruct((N, Cp, mp), jnp.bfloat16),
                   jax.ShapeDtypeStruct((NB, Cp, 1), jnp.float32),
                   jax.ShapeDtypeStruct((NB, Cp, 1), jnp.float32)),
        grid=(NB,),
        in_specs=[pl.BlockSpec((T, Cp, Cs), lambda n: (0, 0, 0)),
                  pl.BlockSpec((gi, Cs, Li), lambda n: (n, 0, 0)),
                  pl.BlockSpec((Cs, 1), lambda n: (0, 0)),
                  pl.BlockSpec((Cs, 1), lambda n: (0, 0)),
                  pl.BlockSpec((mrows, Li), lambda n: (0, 0)),
                  pl.BlockSpec((1, mp), lambda n: (0, 0))],
        out_specs=(pl.BlockSpec((gi, Cp, mp), lambda n: (n, 0, 0)),
                   pl.BlockSpec((1, Cp, 1), lambda n: (n, 0, 0)),
                   pl.BlockSpec((1, Cp, 1), lambda n: (n, 0, 0))),
        scratch_shapes=scratch,
        compiler_params=pltpu.CompilerParams(
            dimension_semantics=("parallel",), vmem_limit_bytes=vmem),
    )(wt, xs, sc, sf, im, om)
    return y, s[:, :, 0], q[:, :, 0]


def _bnrelu_kernel(y_ref, sc_ref, sf_ref, o_ref):
    o_ref[...] = jnp.maximum(
        y_ref[...].astype(jnp.float32) * sc_ref[...] + sf_ref[...], 0.0)


def _bnrelu(y, sc, sf):
    N, Cp, Mp = y.shape
    gi = N // NB
    return pl.pallas_call(
        _bnrelu_kernel,
        out_shape=jax.ShapeDtypeStruct((N, Cp, Mp), jnp.float32),
        grid=(NB,),
        in_specs=[pl.BlockSpec((gi, Cp, Mp), lambda n: (n, 0, 0)),
                  pl.BlockSpec((Cp, 1), lambda n: (0, 0)),
                  pl.BlockSpec((Cp, 1), lambda n: (0, 0))],
        out_specs=pl.BlockSpec((gi, Cp, Mp), lambda n: (n, 0, 0)),
        compiler_params=pltpu.CompilerParams(
            dimension_semantics=("parallel",)),
    )(y, sc.reshape(Cp, 1), sf.reshape(Cp, 1))


def _fc_kernel(x_ref, sc_ref, sf_ref, w1, b1, w2, b2, w3, b3, w4, b4, w5, b5,
               w6, b6, h_ref, d_ref):
    xr = x_ref[...].astype(jnp.float32)
    z = jnp.maximum(xr * sc_ref[...] + sf_ref[...], 0.0)

    def lin(v, w, b):
        return jnp.dot(v.astype(jnp.bfloat16), w[...],
                       preferred_element_type=jnp.float32) + b[...]

    z = jnp.maximum(lin(z, w1, b1), 0.0)
    z = jnp.maximum(lin(z, w2, b2), 0.0)
    h = jax.nn.sigmoid(lin(z, w3, b3))
    h_ref[...] = h
    d = jnp.maximum(lin(h, w4, b4), 0.0)
    d = jnp.maximum(lin(d, w5, b5), 0.0)
    d = jnp.maximum(lin(d, w6, b6), 0.0)
    d_ref[...] = d


def _fc_stack(flat_raw, sc_flat, sf_flat, p):
    Nb, F = flat_raw.shape
    names = ["en_lin1", "en_lin2", "en_lin3", "de_lin3", "de_lin2", "de_lin1"]
    args = [flat_raw, sc_flat, sf_flat]
    for n in names:
        args.append(p[n + "_w"].T.astype(jnp.bfloat16))
        args.append(p[n + "_b"].reshape(1, -1).astype(jnp.float32))
    h_dim = p["en_lin3_w"].shape[0]
    gb = Nb // 2
    specs = [pl.BlockSpec((gb, F), lambda n: (n, 0)),
             pl.BlockSpec((1, F), lambda n: (0, 0)),
             pl.BlockSpec((1, F), lambda n: (0, 0))]
    for a in args[3:]:
        specs.append(pl.BlockSpec(a.shape, lambda n: (0,) * a.ndim))
    h, d = pl.pallas_call(
        _fc_kernel,
        out_shape=(jax.ShapeDtypeStruct((Nb, h_dim), jnp.float32),
                   jax.ShapeDtypeStruct((Nb, F), jnp.float32)),
        grid=(2,),
        in_specs=specs,
        out_specs=(pl.BlockSpec((gb, h_dim), lambda n: (n, 0)),
                   pl.BlockSpec((gb, F), lambda n: (n, 0))),
        compiler_params=pltpu.CompilerParams(
            dimension_semantics=("parallel",)),
    )(*args)
    return h, d


def _fold_conv_weight(w, sh, sw):
    C_out, C_in, kd, kh, kw = w.shape
    Uh, Uw = _cdiv(kh, sh), _cdiv(kw, sw)
    wp = jnp.pad(w, ((0, 0), (0, 0), (0, 0), (0, Uh * sh - kh), (0, Uw * sw - kw)))
    wp = wp.reshape(C_out, C_in, kd, Uh, sh, Uw, sw)
    wp = wp.transpose(2, 3, 5, 0, 1, 4, 6)
    return wp.reshape(kd * Uh * Uw, C_out, C_in * sh * sw)


def _tconv_axis_plan(k, s, p, n_out):
    q_cnt = _cdiv(n_out, s)
    j0 = [(k - 1 - p - r) % s for r in range(s)]
    base = [(r + j0[r] - (k - 1) + p) // s for r in range(s)]
    L = [(k - 1 - j0[r]) // s + 1 for r in range(s)]
    bmin = min(base)
    U = max(base[r] - bmin + L[r] for r in range(s))
    padL = -bmin
    J = np.full((s, U), -1, dtype=np.int64)
    for r in range(s):
        off = base[r] - bmin
        for u in range(L[r]):
            J[r, off + u] = k - 1 - j0[r] - s * u
    return padL, U, J, q_cnt


def _expand_tconv_weight(w, Jd, Jh, Jw):
    sd, Ud = Jd.shape
    sh, Uh = Jh.shape
    sw, Uw = Jw.shape
    jd = Jd.T.reshape(Ud, 1, 1, sd, 1, 1)
    jh = Jh.T.reshape(1, Uh, 1, 1, sh, 1)
    jw = Jw.T.reshape(1, 1, Uw, 1, 1, sw)
    jd, jh, jw = np.broadcast_arrays(jd, jh, jw)
    valid = (jd >= 0) & (jh >= 0) & (jw >= 0)
    wp = w.transpose(1, 0, 2, 3, 4)
    g = wp[:, :, np.maximum(jd, 0), np.maximum(jh, 0), np.maximum(jw, 0)]
    g = g * jnp.asarray(valid, wp.dtype)
    g = g.transpose(2, 3, 4, 0, 5, 6, 7, 1)
    return g.reshape(Ud * Uh * Uw, g.shape[3] * sd * sh * sw, g.shape[-1])


def _bn_affine(ssum, qsum, count, gamma, beta, eps=1e-5):
    mean = ssum / count
    var = jnp.maximum(qsum / count - mean * mean, 0.0)
    scale = gamma.astype(jnp.float32) / jnp.sqrt(var + eps)
    shift = beta.astype(jnp.float32) - mean * scale
    return scale, shift


def _valid_mask(grid_dims, valid_dims, mp):
    dg, hg, wg = grid_dims
    do, ho, wo = valid_dims
    m = np.zeros((dg, hg, wg), np.float32)
    m[:do, :ho, :wo] = 1.0
    out = np.zeros((1, mp), np.float32)
    out[0, :m.size] = m.reshape(-1)
    return jnp.asarray(out)


def _conv_in_mask(C_in, sh, sw, grid_dims, pads, in_dims, li):
    Dg, Hq, Wq = grid_dims
    pd, ph, pw = pads
    D, H, W = in_dims
    d = np.arange(Dg).reshape(-1, 1, 1)
    h = np.arange(Hq).reshape(1, -1, 1)
    w = np.arange(Wq).reshape(1, 1, -1)
    rows = []
    for a in range(sh):
        for b in range(sw):
            ok = ((d >= pd) & (d < pd + D)
                  & (h * sh + a >= ph) & (h * sh + a < ph + H)
                  & (w * sw + b >= pw) & (w * sw + b < pw + W))
            rows.append(ok.reshape(-1))
    m = np.stack(rows).astype(np.float32)
    out = np.zeros((C_in * sh * sw, li), np.float32)
    out[:, :m.shape[1]] = np.tile(m, (C_in, 1))
    return jnp.asarray(out)


def _deconv_in_mask(grid_dims, padl, in_dims, li):
    Dg, Hg, Wg = grid_dims
    pld, plh, plw = padl
    Di, Hi, Wi = in_dims
    m = np.zeros((Dg, Hg, Wg), np.float32)
    m[pld:pld + Di, plh:plh + Hi, plw:plw + Wi] = 1.0
    out = np.zeros((1, li), np.float32)
    out[0, :m.size] = m.reshape(-1)
    return jnp.asarray(out)


ENC_CFG = [("en_conv1", "en_norm1", (1, 3, 3), (0, 2, 2)),
           ("en_conv2", "en_norm2", (1, 2, 2), (0, 2, 2)),
           ("en_conv3", "en_norm3", (1, 2, 2), (0, 2, 2)),
           ("en_conv4", "en_norm4", (1, 2, 2), (0, 2, 2))]
DEC_CFG = [("de_conv4", "de_norm4", (1, 2, 2), (0, 2, 2)),
           ("de_conv3", "de_norm3", (1, 2, 2), (0, 2, 2)),
           ("de_conv2", "de_norm2", (1, 2, 2), (0, 2, 2)),
           ("de_conv1", "de_norm1", (1, 3, 3), (0, 2, 2))]


def _enc_layer(x5_raw, aff, w, gamma, beta, stride, pad):
    N, C_in, D, H, W = x5_raw.shape
    C_out, _, kd, kh, kw = w.shape
    _, sh, sw = stride
    pd, ph, pw = pad
    Do = D + 2 * pd - kd + 1
    Ho = (H + 2 * ph - kh) // sh + 1
    Wo = (W + 2 * pw - kw) // sw + 1
    Uh, Uw = _cdiv(kh, sh), _cdiv(kw, sw)
    Dg = D + 2 * pd
    Hq = max(_cdiv(H + 2 * ph, sh), Ho + Uh - 1)
    Wq = max(_cdiv(W + 2 * pw, sw), Wo + Uw - 1)

    Mv = Dg * Hq * Wq

    def _fold(chunk):
        cpad = jnp.pad(chunk, ((0, 0), (0, 0), (pd, Dg - D - pd),
                               (ph, Hq * sh - H - ph), (pw, Wq * sw - W - pw)))
        cs = cpad.reshape(-1, C_in, Dg, Hq, sh, Wq, sw).transpose(0, 1, 4, 6, 2, 3, 5)
        return cs.reshape(-1, C_in * sh * sw, Mv)

    xs = _fold(x5_raw)

    offsets = tuple(td * Hq * Wq + th * Wq + tw
                    for td in range(kd) for th in range(Uh) for tw in range(Uw))
    Mp = _round_up(Mv, 128)
    Li = Mp + (_round_up(offsets[-1], 128) if offsets[-1] else 0)
    xs = jnp.pad(xs, ((0, 0), (0, 0), (0, Li - Mv)))

    wt = _fold_conv_weight(w.astype(jnp.bfloat16), sh, sw)
    Cp = _round_up(C_out, 8)
    wt = jnp.pad(wt, ((0, 0), (0, Cp - C_out), (0, 0)))
    Cs = C_in * sh * sw

    if aff is not None:
        sc = jnp.repeat(aff[0], sh * sw).reshape(Cs, 1)
        sf = jnp.repeat(aff[1], sh * sw).reshape(Cs, 1)
        im = _conv_in_mask(C_in, sh, sw, (Dg, Hq, Wq), pad, (D, H, W), Li)
    else:
        sc = jnp.ones((Cs, 1), jnp.float32)
        sf = jnp.zeros((Cs, 1), jnp.float32)
        im = jnp.ones((1, Li), jnp.float32)
    om = _valid_mask((Dg, Hq, Wq), (Do, Ho, Wo), Mp)

    y, s, q = _tap_layer(wt, xs, sc, sf, im, om, Mp, offsets, aff is not None)
    cnt = float(N * Do * Ho * Wo)
    scale, shift = _bn_affine(jnp.sum(s[:, :C_out], 0), jnp.sum(q[:, :C_out], 0),
                              cnt, gamma, beta)
    y5_raw = (y[:, :C_out, :Mv].reshape(N, C_out, Dg, Hq, Wq)
              [:, :, :Do, :Ho, :Wo])
    return y5_raw, (scale, shift)


def _dec_layer(x5_raw, aff, w, gamma, beta, stride, pad):
    N, C_in, Di, Hi, Wi = x5_raw.shape
    _, C_out, kd, kh, kw = w.shape
    sd, sh, sw = stride
    pd, ph, pw = pad
    Do = (Di - 1) * sd - 2 * pd + kd
    Ho = (Hi - 1) * sh - 2 * ph + kh
    Wo = (Wi - 1) * sw - 2 * pw + kw
    pld, Ud, Jd, Qd = _tconv_axis_plan(kd, sd, pd, Do)
    plh, Uh, Jh, Qh = _tconv_axis_plan(kh, sh, ph, Ho)
    plw, Uw, Jw, Qw = _tconv_axis_plan(kw, sw, pw, Wo)
    Dg, Hg, Wg = Qd + Ud - 1, Qh + Uh - 1, Qw + Uw - 1

    xpad = jnp.pad(x5_raw, ((0, 0), (0, 0), (pld, Dg - Di - pld),
                            (plh, Hg - Hi - plh), (plw, Wg - Wi - plw)))
    Mv = Dg * Hg * Wg
    xs = xpad.reshape(N, C_in, Mv)

    offsets = tuple(td * Hg * Wg + th * Wg + tw
                    for td in range(Ud) for th in range(Uh) for tw in range(Uw))
    Mp = _round_up(Mv, 128)
    Li = Mp + (_round_up(offsets[-1], 128) if offsets[-1] else 0)
    xs = jnp.pad(xs, ((0, 0), (0, 0), (0, Li - Mv)))

    nph = sd * sh * sw
    wt = _expand_tconv_weight(w.astype(jnp.bfloat16), Jd, Jh, Jw)
    Cph = C_out * nph
    Cp = _round_up(Cph, 8)
    wt = jnp.pad(wt, ((0, 0), (0, Cp - Cph), (0, 0)))

    if aff is not None:
        sc = aff[0].reshape(C_in, 1)
        sf = aff[1].reshape(C_in, 1)
        im = _deconv_in_mask((Dg, Hg, Wg), (pld, plh, plw), (Di, Hi, Wi), Li)
    else:
        sc = jnp.ones((C_in, 1), jnp.float32)
        sf = jnp.zeros((C_in, 1), jnp.float32)
        im = jnp.ones((1, Li), jnp.float32)
    om = _valid_mask((Dg, Hg, Wg), (Qd, Qh, Qw), Mp)

    y, s, q = _tap_layer(wt, xs, sc, sf, im, om, Mp, offsets, aff is not None)
    cnt = float(N * Do * Ho * Wo)
    ssum = jnp.sum(s[:, :Cph], 0).reshape(C_out, nph).sum(1)
    qsum = jnp.sum(q[:, :Cph], 0).reshape(C_out, nph).sum(1)
    scale, shift = _bn_affine(ssum, qsum, cnt, gamma, beta)
    geom = (Cph, Mv, C_out, (sd, sh, sw), (Dg, Hg, Wg), (Qd, Qh, Qw),
            (Do, Ho, Wo))
    return y, (scale, shift), geom


def _dec_to5d(y, geom, chunks=1):
    N = y.shape[0]
    Cph, Mv, C_out, (sd, sh, sw), (Dg, Hg, Wg), (Qd, Qh, Qw), (Do, Ho, Wo) = geom

    def _d2s(yc):
        y8 = yc[:, :Cph, :Mv].reshape(-1, C_out, sd, sh, sw, Dg, Hg, Wg)
        y8 = y8[:, :, :, :, :, :Qd, :Qh, :Qw]
        y8 = y8.transpose(0, 1, 5, 2, 6, 3, 7, 4)
        return y8.reshape(-1, C_out, Qd * sd, Qh * sh, Qw * sw)[:, :, :Do, :Ho, :Wo]

    if chunks > 1 and N % chunks == 0:
        g = N // chunks
        return jnp.concatenate([_d2s(y[i:i + g]) for i in range(0, N, g)], axis=0)
    return _d2s(y)


def kernel(x, en_conv1_w, en_norm1_g, en_norm1_b, en_conv2_w, en_norm2_g, en_norm2_b, en_conv3_w, en_norm3_g, en_norm3_b, en_conv4_w, en_norm4_g, en_norm4_b, de_conv4_w, de_norm4_g, de_norm4_b, de_conv3_w, de_norm3_g, de_norm3_b, de_conv2_w, de_norm2_g, de_norm2_b, de_conv1_w, de_norm1_g, de_norm1_b, en_lin1_w, en_lin1_b, en_lin2_w, en_lin2_b, en_lin3_w, en_lin3_b, de_lin3_w, de_lin3_b, de_lin2_w, de_lin2_b, de_lin1_w, de_lin1_b):
    p = {
        "en_conv1_w": en_conv1_w, "en_norm1_g": en_norm1_g, "en_norm1_b": en_norm1_b,
        "en_conv2_w": en_conv2_w, "en_norm2_g": en_norm2_g, "en_norm2_b": en_norm2_b,
        "en_conv3_w": en_conv3_w, "en_norm3_g": en_norm3_g, "en_norm3_b": en_norm3_b,
        "en_conv4_w": en_conv4_w, "en_norm4_g": en_norm4_g, "en_norm4_b": en_norm4_b,
        "de_conv4_w": de_conv4_w, "de_norm4_g": de_norm4_g, "de_norm4_b": de_norm4_b,
        "de_conv3_w": de_conv3_w, "de_norm3_g": de_norm3_g, "de_norm3_b": de_norm3_b,
        "de_conv2_w": de_conv2_w, "de_norm2_g": de_norm2_g, "de_norm2_b": de_norm2_b,
        "de_conv1_w": de_conv1_w, "de_norm1_g": de_norm1_g, "de_norm1_b": de_norm1_b,
        "en_lin1_w": en_lin1_w, "en_lin1_b": en_lin1_b,
        "en_lin2_w": en_lin2_w, "en_lin2_b": en_lin2_b,
        "en_lin3_w": en_lin3_w, "en_lin3_b": en_lin3_b,
        "de_lin3_w": de_lin3_w, "de_lin3_b": de_lin3_b,
        "de_lin2_w": de_lin2_w, "de_lin2_b": de_lin2_b,
        "de_lin1_w": de_lin1_w, "de_lin1_b": de_lin1_b,
    }
    N = x.shape[0]
    cur5 = x.astype(jnp.bfloat16)
    aff = None
    for conv, norm, stride, pad in ENC_CFG:
        cur5, aff = _enc_layer(cur5, aff, p[conv + "_w"], p[norm + "_g"],
                               p[norm + "_b"], stride, pad)

    enc_size = cur5.shape
    flat_raw = cur5.reshape(N, -1)
    rep = enc_size[2] * enc_size[3] * enc_size[4]
    sc_flat = jnp.repeat(aff[0], rep).reshape(1, -1)
    sf_flat = jnp.repeat(aff[1], rep).reshape(1, -1)
    h, d = _fc_stack(flat_raw, sc_flat, sf_flat, p)

    cur5 = d.reshape(enc_size).astype(jnp.bfloat16)
    aff = None
    for li, (conv, norm, stride, pad) in enumerate(DEC_CFG):
        y, aff_new, geom = _dec_layer(cur5, aff, p[conv + "_w"], p[norm + "_g"],
                                      p[norm + "_b"], stride, pad)
        if li < len(DEC_CFG) - 1:
            cur5 = _dec_to5d(y, geom)
            aff = aff_new
        else:
            Cph, nph = geom[0], geom[3][0] * geom[3][1] * geom[3][2]
            Cp = y.shape[1]
            sc_ph = jnp.pad(jnp.repeat(aff_new[0], nph), (0, Cp - Cph))
            sf_ph = jnp.pad(jnp.repeat(aff_new[1], nph), (0, Cp - Cph))
            o = _bnrelu(y, sc_ph, sf_ph)
            recon = _dec_to5d(o, geom)
    return h, recon
